# baseline (speedup 1.0000x reference)
import functools

import jax
import jax.numpy as jnp
from jax import lax
from jax.experimental import pallas as pl
from jax.experimental.pallas import tpu as pltpu
from jax.experimental.pallas import tpu_sc as plsc

_COLS = 1_000_000
_CH = 1024
_NCH = 976
_SLOTS = 8
_K = 4
_L = 2


def _sc_copy(xt):
    mesh = plsc.VectorSubcoreMesh(core_axis_name="c", subcore_axis_name="s")

    @functools.partial(
        pl.kernel,
        mesh=mesh,
        out_type=jax.ShapeDtypeStruct(xt.shape, xt.dtype),
        scratch_types=[
            pltpu.VMEM((_K, 8, _CH), jnp.float32),
            pltpu.SemaphoreType.DMA((_K,)),
            pltpu.SemaphoreType.DMA((_K,)),
        ],
        compiler_params=pltpu.CompilerParams(use_tc_tiling_on_sc=True),
    )
    def copy_kernel(in_hbm, out_hbm, bufs, rsem, wsem):
        nc = 2
        wid = lax.axis_index("s") * nc + lax.axis_index("c")
        g = lax.rem(wid, 4)
        l = lax.div(wid, 4)
        row0 = pl.multiple_of(g * 8, 8)
        n_j = jnp.where(l == 0, (_NCH + _SLOTS - 1) // _SLOTS,
                        _NCH // _SLOTS)

        def col(j):
            return pl.multiple_of((l + _SLOTS * j) * _CH, 128)

        def rd(j, s):
            return pltpu.make_async_copy(
                in_hbm.at[pl.ds(row0, 8), pl.ds(col(j), _CH)],
                bufs.at[s], rsem.at[s])

        def wr(j, s):
            return pltpu.make_async_copy(
                bufs.at[s], out_hbm.at[pl.ds(row0, 8), pl.ds(col(j), _CH)],
                wsem.at[s])

        for j in range(_L):
            rd(jnp.int32(j), j).start()

        def body(j, _):
            s = lax.rem(j, _K)
            rd(j, s).wait()
            wr(j, s).start()
            jn = j + _L
            sn = lax.rem(jn, _K)

            @pl.when(jnp.logical_and(jn < n_j, jn >= _K))
            def _():
                wr(jn - _K, sn).wait()

            @pl.when(jn < n_j)
            def _():
                rd(jn, sn).start()

            return 0

        lax.fori_loop(0, n_j, body, 0)

        def drain(j, _):
            wr(j, lax.rem(j, _K)).wait()
            return 0

        lax.fori_loop(n_j - _K, n_j, drain, 0)

        @pl.when(l == 0)
        def _():
            tcol = pl.multiple_of(_NCH * _CH, 128)
            tin = pltpu.make_async_copy(
                in_hbm.at[pl.ds(row0, 8), pl.ds(tcol, 512)],
                bufs.at[0, :, :512], rsem.at[0])
            tin.start()
            tin.wait()
            tout = pltpu.make_async_copy(
                bufs.at[0, :, :512],
                out_hbm.at[pl.ds(row0, 8), pl.ds(tcol, 512)], wsem.at[0])
            tout.start()
            tout.wait()

    return copy_kernel(xt)


def _edge_block(in_ref, alias_ref, out_ref):
    del alias_ref
    out_ref[...] = in_ref[...]


def _tc_edge_patch(xt, partial):
    spec = pl.BlockSpec((32, 128), lambda i: (0, _COLS // 128))
    return pl.pallas_call(
        _edge_block,
        grid=(1,),
        in_specs=[spec, pl.BlockSpec(memory_space=pl.ANY)],
        out_specs=spec,
        out_shape=jax.ShapeDtypeStruct(xt.shape, xt.dtype),
        input_output_aliases={1: 0},
    )(xt, partial)


def kernel(item_emb):
    xt = item_emb.T
    out_t = _tc_edge_patch(xt, _sc_copy(xt))
    return out_t.T


# speedup vs baseline: 1.0395x; 1.0395x over previous
import functools

import jax
import jax.numpy as jnp
from jax import lax
from jax.experimental import pallas as pl
from jax.experimental.pallas import tpu as pltpu
from jax.experimental.pallas import tpu_sc as plsc

_COLS = 1_000_000
_CH = 512
_NCH = 1953
_SLOTS = 8
_K = 8
_L = 4


def _sc_copy(xt):
    mesh = plsc.VectorSubcoreMesh(core_axis_name="c", subcore_axis_name="s")

    @functools.partial(
        pl.kernel,
        mesh=mesh,
        out_type=jax.ShapeDtypeStruct(xt.shape, xt.dtype),
        scratch_types=[
            pltpu.VMEM((_K, 8, _CH), jnp.float32),
            pltpu.SemaphoreType.DMA((_K,)),
            pltpu.SemaphoreType.DMA((_K,)),
        ],
        compiler_params=pltpu.CompilerParams(use_tc_tiling_on_sc=True),
    )
    def copy_kernel(in_hbm, out_hbm, bufs, rsem, wsem):
        nc = 2
        wid = lax.axis_index("s") * nc + lax.axis_index("c")
        g = lax.rem(wid, 4)
        l = lax.div(wid, 4)
        row0 = pl.multiple_of(g * 8, 8)
        n_j = jnp.where(l == 0, (_NCH + _SLOTS - 1) // _SLOTS,
                        _NCH // _SLOTS)

        def col(j):
            return pl.multiple_of((l + _SLOTS * j) * _CH, 128)

        def rd(j, s):
            return pltpu.make_async_copy(
                in_hbm.at[pl.ds(row0, 8), pl.ds(col(j), _CH)],
                bufs.at[s], rsem.at[s])

        def wr(j, s):
            return pltpu.make_async_copy(
                bufs.at[s], out_hbm.at[pl.ds(row0, 8), pl.ds(col(j), _CH)],
                wsem.at[s])

        for j in range(_L):
            rd(jnp.int32(j), j).start()

        def body(j, _):
            s = lax.rem(j, _K)
            rd(j, s).wait()
            wr(j, s).start()
            jn = j + _L
            sn = lax.rem(jn, _K)

            @pl.when(jnp.logical_and(jn < n_j, jn >= _K))
            def _():
                wr(jn - _K, sn).wait()

            @pl.when(jn < n_j)
            def _():
                rd(jn, sn).start()

            return 0

        lax.fori_loop(0, n_j, body, 0)

        def drain(j, _):
            wr(j, lax.rem(j, _K)).wait()
            return 0

        lax.fori_loop(n_j - _K, n_j, drain, 0)

    return copy_kernel(xt)


def _edge_block(in_ref, alias_ref, out_ref):
    del alias_ref
    out_ref[...] = in_ref[...]


def _tc_edge_patch(xt, partial):
    spec = pl.BlockSpec((32, 128), lambda i: (0, _COLS // 128))
    return pl.pallas_call(
        _edge_block,
        grid=(1,),
        in_specs=[spec, pl.BlockSpec(memory_space=pl.ANY)],
        out_specs=spec,
        out_shape=jax.ShapeDtypeStruct(xt.shape, xt.dtype),
        input_output_aliases={1: 0},
    )(xt, partial)


def kernel(item_emb):
    xt = item_emb.T
    out_t = _tc_edge_patch(xt, _sc_copy(xt))
    return out_t.T


# speedup vs baseline: 1.0665x; 1.0260x over previous
import functools

import jax
import jax.numpy as jnp
from jax import lax
from jax.experimental import pallas as pl
from jax.experimental.pallas import tpu as pltpu
from jax.experimental.pallas import tpu_sc as plsc

_COLS = 1_000_000
_CH = 512
_NCH = 1953
_SLOTS = 8
_K = 8
_L = 6


def _sc_copy(xt):
    mesh = plsc.VectorSubcoreMesh(core_axis_name="c", subcore_axis_name="s")

    @functools.partial(
        pl.kernel,
        mesh=mesh,
        out_type=jax.ShapeDtypeStruct(xt.shape, xt.dtype),
        scratch_types=[
            pltpu.VMEM((_K, 8, _CH), jnp.float32),
            pltpu.SemaphoreType.DMA((_K,)),
            pltpu.SemaphoreType.DMA((_K,)),
        ],
        compiler_params=pltpu.CompilerParams(use_tc_tiling_on_sc=True),
    )
    def copy_kernel(in_hbm, out_hbm, bufs, rsem, wsem):
        nc = 2
        wid = lax.axis_index("s") * nc + lax.axis_index("c")
        g = lax.rem(wid, 4)
        l = lax.div(wid, 4)
        row0 = pl.multiple_of(g * 8, 8)
        n_j = jnp.where(l == 0, (_NCH + _SLOTS - 1) // _SLOTS,
                        _NCH // _SLOTS)

        def col(j):
            return pl.multiple_of((l + _SLOTS * j) * _CH, 128)

        def rd(j, s):
            return pltpu.make_async_copy(
                in_hbm.at[pl.ds(row0, 8), pl.ds(col(j), _CH)],
                bufs.at[s], rsem.at[s])

        def wr(j, s):
            return pltpu.make_async_copy(
                bufs.at[s], out_hbm.at[pl.ds(row0, 8), pl.ds(col(j), _CH)],
                wsem.at[s])

        for j in range(_L):
            rd(jnp.int32(j), j).start()

        def body(j, _):
            s = lax.rem(j, _K)
            rd(j, s).wait()
            wr(j, s).start()
            jn = j + _L
            sn = lax.rem(jn, _K)

            @pl.when(jnp.logical_and(jn < n_j, jn >= _K))
            def _():
                wr(jn - _K, sn).wait()

            @pl.when(jn < n_j)
            def _():
                rd(jn, sn).start()

            return 0

        lax.fori_loop(0, n_j, body, 0)

        def drain(j, _):
            wr(j, lax.rem(j, _K)).wait()
            return 0

        lax.fori_loop(n_j - _K, n_j, drain, 0)

    return copy_kernel(xt)


def _edge_block(in_ref, alias_ref, out_ref):
    del alias_ref
    out_ref[...] = in_ref[...]


def _tc_edge_patch(xt, partial):
    spec = pl.BlockSpec((32, 128), lambda i: (0, _COLS // 128))
    return pl.pallas_call(
        _edge_block,
        grid=(1,),
        in_specs=[spec, pl.BlockSpec(memory_space=pl.ANY)],
        out_specs=spec,
        out_shape=jax.ShapeDtypeStruct(xt.shape, xt.dtype),
        input_output_aliases={1: 0},
    )(xt, partial)


def kernel(item_emb):
    xt = item_emb.T
    out_t = _tc_edge_patch(xt, _sc_copy(xt))
    return out_t.T
